# Initial kernel scaffold; baseline (speedup 1.0000x reference)
import jax
import jax.numpy as jnp
from jax.experimental import pallas as pl


def kernel(atom_embed, atom_res_idx, fastpass, ln_gamma, ln_beta, W_proj, W_out, b_out):
    raise NotImplementedError("write your pallas kernel here")


# speedup vs baseline: 7.1164x; 7.1164x over previous
import jax
import jax.numpy as jnp
from jax.experimental import pallas as pl
from jax.experimental.pallas import tpu as pltpu

_NRES = 4096
_NAA = 21
_BLK = 2048
_W = 128


def _seg_body(idx_ref, x_ref, g_ref, bta_ref, wpt_ref, wot_ref, bo_ref,
              out_ref, acc_ref, cnt_ref):
    j = pl.program_id(1)
    nblk = pl.num_programs(1)

    @pl.when(j == 0)
    def _init():
        acc_ref[...] = jnp.zeros_like(acc_ref)
        cnt_ref[...] = jnp.zeros_like(cnt_ref)

    x = x_ref[0]
    mu = jnp.mean(x, axis=-1, keepdims=True)
    xc = x - mu
    var = jnp.mean(xc * xc, axis=-1, keepdims=True)
    y = xc * jax.lax.rsqrt(var + 1e-5) * g_ref[...] + bta_ref[...]
    yb = y.astype(jnp.bfloat16)

    idx = idx_ref[0, 0]
    r_first = idx_ref[0, 0, 0, 0]
    r_last = idx_ref[0, 0, 0, _BLK - 1]
    w0 = r_first // _W
    nwin = r_last // _W - w0 + 1

    def body(w, carry):
        base = (w0 + w) * _W
        cols = base + jax.lax.broadcasted_iota(jnp.int32, (_W, 1), 0)
        mask = cols == idx
        oh = mask.astype(jnp.bfloat16)
        part = jax.lax.dot(oh, yb, preferred_element_type=jnp.float32)
        acc_ref[pl.ds(base, _W), :] += part
        cnt = jnp.sum(mask.astype(jnp.float32), axis=1, keepdims=True)
        cnt_ref[pl.ds(base, _W), :] += cnt
        return carry

    jax.lax.fori_loop(0, nwin, body, 0)

    @pl.when(j == nblk - 1)
    def _epilogue():
        s = acc_ref[...]
        t = jax.lax.dot(s, wpt_ref[...], preferred_element_type=jnp.float32)
        t = t / (cnt_ref[...] + 1.0)
        o = jax.lax.dot(t, wot_ref[...], preferred_element_type=jnp.float32)
        out_ref[0] = o + bo_ref[...]


def kernel(atom_embed, atom_res_idx, fastpass, ln_gamma, ln_beta,
           W_proj, W_out, b_out):
    del fastpass
    b, n, c = atom_embed.shape
    nblk = n // _BLK
    idxr = atom_res_idx.astype(jnp.int32).reshape(b, nblk, 1, _BLK)
    g = ln_gamma.reshape(1, c).astype(jnp.float32)
    bta = ln_beta.reshape(1, c).astype(jnp.float32)
    wpt = W_proj.T.astype(jnp.float32)
    wot = jnp.zeros((c, c), jnp.float32).at[:, :_NAA].set(W_out.T)
    bo = jnp.zeros((1, c), jnp.float32).at[0, :_NAA].set(b_out)

    out = pl.pallas_call(
        _seg_body,
        grid=(b, nblk),
        in_specs=[
            pl.BlockSpec((1, 1, 1, _BLK), lambda bi, ji: (bi, ji, 0, 0)),
            pl.BlockSpec((1, _BLK, c), lambda bi, ji: (bi, ji, 0)),
            pl.BlockSpec((1, c), lambda bi, ji: (0, 0)),
            pl.BlockSpec((1, c), lambda bi, ji: (0, 0)),
            pl.BlockSpec((c, c), lambda bi, ji: (0, 0)),
            pl.BlockSpec((c, c), lambda bi, ji: (0, 0)),
            pl.BlockSpec((1, c), lambda bi, ji: (0, 0)),
        ],
        out_specs=pl.BlockSpec((1, _NRES, c), lambda bi, ji: (bi, 0, 0)),
        out_shape=jax.ShapeDtypeStruct((b, _NRES, c), jnp.float32),
        scratch_shapes=[
            pltpu.VMEM((_NRES, c), jnp.float32),
            pltpu.VMEM((_NRES, 1), jnp.float32),
        ],
        compiler_params=pltpu.CompilerParams(
            dimension_semantics=("arbitrary", "arbitrary")),
    )(idxr, atom_embed, g, bta, wpt, wot, bo)
    return out[..., :_NAA]
